# baseline (speedup 1.0000x reference)
import functools

import jax
import jax.numpy as jnp
from jax import lax
from jax.experimental import pallas as pl
from jax.experimental.pallas import tpu as pltpu
from jax.experimental.pallas import tpu_sc as plsc

B, S, DIM = 4, 8192, 128
N = B * S

_info = plsc.get_sparse_core_info()
NC, NS, L = _info.num_cores, _info.num_subcores, _info.num_lanes
NW = NC * NS
C = N // NW
BLK = 32
NBMAX = C // BLK
CAP = C + L
NBUF = 4
DEPTH = 3
NDUMP = NW * 4 * BLK

_mesh = plsc.VectorSubcoreMesh(core_axis_name="c", subcore_axis_name="s")


@functools.partial(
    pl.kernel,
    mesh=_mesh,
    compiler_params=pltpu.CompilerParams(needs_layout_passes=False),
    out_type=jax.ShapeDtypeStruct((N + NDUMP, DIM), jnp.float32),
    scratch_types=(
        [pltpu.VMEM((C,), jnp.int32), pltpu.VMEM((C,), jnp.int32)]
        + [pltpu.VMEM((CAP,), jnp.int32) for _ in range(4)]
        + [pltpu.VMEM((CAP,), jnp.int32) for _ in range(4)]
        + [pltpu.VMEM((NBMAX, BLK), jnp.int32) for _ in range(4)]
        + [pltpu.VMEM((BLK, DIM), jnp.float32) for _ in range(NBUF)]
        + [pltpu.SemaphoreType.DMA for _ in range(2 * NBUF)]
    ),
)
def _sc_lookup(ids_hbm, mods_hbm, t0, t1, t2, t3, out_hbm, *scratch):
    ids_v, mods_v = scratch[0], scratch[1]
    idl = scratch[2:6]
    posl = scratch[6:10]
    pos2d = scratch[10:14]
    gbufs = scratch[14 : 14 + NBUF]
    gsems = scratch[14 + NBUF : 14 + 2 * NBUF]
    ssems = scratch[14 + 2 * NBUF :]
    tables = (t0, t1, t2, t3)

    wid = lax.axis_index("s") * NC + lax.axis_index("c")
    base = wid * C
    pltpu.sync_copy(ids_hbm.at[pl.ds(base, C)], ids_v)
    pltpu.sync_copy(mods_hbm.at[pl.ds(base, C)], mods_v)

    zed = jnp.zeros((L,), jnp.int32)
    for t in range(4):
        dump_base = N + (wid * 4 + t) * BLK
        dv = [jnp.arange(L, dtype=jnp.int32) + (dump_base + g * L) for g in range(BLK // L)]
        for i in range(CAP // L):
            idl[t][pl.ds(i * L, L)] = zed
            posl[t][pl.ds(i * L, L)] = dv[i % (BLK // L)]

    offs = [jnp.int32(0)] * 4
    for i in range(C // L):
        sl = pl.ds(i * L, L)
        iv = ids_v[sl]
        mv = mods_v[sl]
        pv = jnp.arange(L, dtype=jnp.int32) + (base + i * L)
        for t in range(4):
            m = mv == t
            mi = m.astype(jnp.int32)
            dst = plsc.cumsum(mi) + (offs[t] - 1)
            plsc.store_scatter(idl[t], [dst], iv, mask=m)
            plsc.store_scatter(posl[t], [dst], pv, mask=m)
            offs[t] = offs[t] + jnp.max(plsc.all_reduce_population_count(m))

    for t in range(4):
        for b in range(NBMAX):
            for h in range(BLK // L):
                pos2d[t][b, pl.ds(h * L, L)] = posl[t][pl.ds(b * BLK + h * L, L)]

    blocks = [(t, b) for t in range(4) for b in range(NBMAX)]
    conds = {}
    for j in range(len(blocks) + DEPTH):
        if j < len(blocks):
            t, b = blocks[j]
            u = j % NBUF
            cond = jnp.int32(b * BLK) < offs[t]
            conds[j] = cond
            jp = j - NBUF
            if jp >= 0:
                tp, bp = blocks[jp]

                @pl.when(conds[jp])
                def _(tp=tp, bp=bp, u=u):
                    pltpu.make_async_copy(
                        gbufs[u], out_hbm.at[pos2d[tp].at[bp]], ssems[u]
                    ).wait()

            @pl.when(cond)
            def _(t=t, b=b, u=u):
                pltpu.async_copy(
                    tables[t].at[idl[t].at[pl.ds(b * BLK, BLK)]], gbufs[u], gsems[u]
                )
        jj = j - DEPTH
        if jj >= 0:
            t2, b2 = blocks[jj]
            u2 = jj % NBUF

            @pl.when(conds[jj])
            def _(t2=t2, b2=b2, u2=u2):
                pltpu.make_async_copy(
                    tables[t2].at[idl[t2].at[pl.ds(b2 * BLK, BLK)]], gbufs[u2], gsems[u2]
                ).wait()
                pltpu.async_copy(gbufs[u2], out_hbm.at[pos2d[t2].at[b2]], ssems[u2])

    for j in range(max(0, len(blocks) - NBUF), len(blocks)):
        t, b = blocks[j]
        u = j % NBUF

        @pl.when(conds[j])
        def _(t=t, b=b, u=u):
            pltpu.make_async_copy(
                gbufs[u], out_hbm.at[pos2d[t].at[b]], ssems[u]
            ).wait()


def kernel(input_ids, modality_ids, text_table, image_table, video_table, audio_table):
    ids = input_ids.reshape(-1)
    mods = modality_ids.reshape(-1)
    out = _sc_lookup(ids, mods, text_table, image_table, video_table, audio_table)
    return out[:N].reshape(B, S, DIM)


# speedup vs baseline: 1.9445x; 1.9445x over previous
import functools

import jax
import jax.numpy as jnp
from jax import lax
from jax.experimental import pallas as pl
from jax.experimental.pallas import tpu as pltpu
from jax.experimental.pallas import tpu_sc as plsc

B, S, DIM = 4, 8192, 128
N = B * S

_info = plsc.get_sparse_core_info()
NC, NS, L = _info.num_cores, _info.num_subcores, _info.num_lanes
NW = NC * NS
C = N // NW
H = C // 2
BLK = 32
BLK_SHIFT = 5
NB = H // BLK
CAP = H + BLK
ROWS = H + 4 * BLK

_mesh = plsc.VectorSubcoreMesh(core_axis_name="c", subcore_axis_name="s")


@functools.partial(
    pl.kernel,
    mesh=_mesh,
    compiler_params=pltpu.CompilerParams(needs_layout_passes=False),
    out_type=jax.ShapeDtypeStruct((N, DIM), jnp.float32),
    scratch_types=(
        [pltpu.VMEM((C,), jnp.int32), pltpu.VMEM((C,), jnp.int32)]
        + [pltpu.VMEM((CAP,), jnp.int32) for _ in range(4)]
        + [pltpu.VMEM((CAP,), jnp.int32) for _ in range(4)]
        + [pltpu.VMEM((NB, BLK), jnp.int32) for _ in range(4)]
        + [pltpu.VMEM((ROWS, DIM), jnp.float32)]
        + [pltpu.SemaphoreType.DMA, pltpu.SemaphoreType.DMA]
    ),
)
def _sc_lookup(ids_hbm, mods_hbm, t0, t1, t2, t3, out_hbm, *scratch):
    ids_v, mods_v = scratch[0], scratch[1]
    idl = scratch[2:6]
    posl = scratch[6:10]
    pos2d = scratch[10:14]
    rows = scratch[14]
    gsem, ssem = scratch[15], scratch[16]
    tables = (t0, t1, t2, t3)

    wid = lax.axis_index("s") * NC + lax.axis_index("c")
    base = wid * C
    pltpu.sync_copy(ids_hbm.at[pl.ds(base, C)], ids_v)
    pltpu.sync_copy(mods_hbm.at[pl.ds(base, C)], mods_v)

    pending = []

    for h in range(2):
        offs = [jnp.zeros((L,), jnp.int32) for _ in range(4)]
        for i in range(H // L):
            sl = pl.ds(h * H + i * L, L)
            iv = ids_v[sl]
            mv = mods_v[sl]
            pv = jnp.arange(L, dtype=jnp.int32) + (base + h * H + i * L)
            for t in range(4):
                m = mv == t
                dst = plsc.cumsum(m.astype(jnp.int32)) + (offs[t] - 1)
                plsc.store_scatter(idl[t], [dst], iv, mask=m)
                plsc.store_scatter(posl[t], [dst], pv, mask=m)
                offs[t] = offs[t] + plsc.all_reduce_population_count(m)

        cnt = [jnp.max(offs[t]) for t in range(4)]

        for t in range(4):

            @pl.when(cnt[t] > 0)
            def _(t=t):
                last = jnp.broadcast_to(cnt[t] - 1, (L,)).astype(jnp.int32)
                idsp = plsc.load_gather(idl[t], [last])
                possp = plsc.load_gather(posl[t], [last])
                for g in range(BLK // L):
                    fill = cnt[t] + jnp.arange(L, dtype=jnp.int32) + g * L
                    plsc.store_scatter(idl[t], [fill], idsp)
                    plsc.store_scatter(posl[t], [fill], possp)

        nbk = [
            lax.shift_right_logical(cnt[t] + (BLK - 1), BLK_SHIFT) for t in range(4)
        ]
        seg = [None] * 4
        seg[0] = jnp.int32(0)
        for t in range(1, 4):
            seg[t] = seg[t - 1] + lax.shift_left(nbk[t - 1], BLK_SHIFT)

        for cond, dsl, tp, bp in pending:

            @pl.when(cond)
            def _(dsl=dsl, tp=tp, bp=bp):
                pltpu.make_async_copy(
                    rows.at[dsl], out_hbm.at[pos2d[tp].at[bp]], ssem
                ).wait()
        pending = []

        for t in range(4):
            for b in range(NB):
                for g in range(BLK // L):
                    pos2d[t][b, pl.ds(g * L, L)] = posl[t][pl.ds(b * BLK + g * L, L)]

        for t in range(4):
            for b in range(NB):

                @pl.when(b < nbk[t])
                def _(t=t, b=b):
                    pltpu.async_copy(
                        tables[t].at[idl[t].at[pl.ds(b * BLK, BLK)]],
                        rows.at[pl.ds(seg[t] + b * BLK, BLK)],
                        gsem,
                    )
        for t in range(4):
            for b in range(NB):

                @pl.when(b < nbk[t])
                def _(t=t, b=b):
                    pltpu.make_async_copy(
                        tables[t].at[idl[t].at[pl.ds(b * BLK, BLK)]],
                        rows.at[pl.ds(seg[t] + b * BLK, BLK)],
                        gsem,
                    ).wait()

        for t in range(4):
            for b in range(NB):
                cond = b < nbk[t]
                dsl = pl.ds(seg[t] + b * BLK, BLK)

                @pl.when(cond)
                def _(t=t, b=b, dsl=dsl):
                    pltpu.async_copy(rows.at[dsl], out_hbm.at[pos2d[t].at[b]], ssem)

                pending.append((cond, dsl, t, b))

    for cond, dsl, tp, bp in pending:

        @pl.when(cond)
        def _(dsl=dsl, tp=tp, bp=bp):
            pltpu.make_async_copy(
                rows.at[dsl], out_hbm.at[pos2d[tp].at[bp]], ssem
            ).wait()


def kernel(input_ids, modality_ids, text_table, image_table, video_table, audio_table):
    ids = input_ids.reshape(-1)
    mods = modality_ids.reshape(-1)
    out = _sc_lookup(ids, mods, text_table, image_table, video_table, audio_table)
    return out.reshape(B, S, DIM)
